# Initial kernel scaffold; baseline (speedup 1.0000x reference)
import jax
import jax.numpy as jnp
from jax.experimental import pallas as pl


def kernel(score_note_dur, phoneme_seq, phoneme_order, emb_word, emb_pos, mix_w1, mix_b1, mix_w2, mix_b2, l0f_wih, l0f_whh, l0f_bih, l0f_bhh, l0b_wih, l0b_whh, l0b_bih, l0b_bhh, l1f_wih, l1f_whh, l1f_bih, l1f_bhh, l1b_wih, l1b_whh, l1b_bih, l1b_bhh, cnn_w1, cnn_b1, cnn_w2, cnn_b2):
    raise NotImplementedError("write your pallas kernel here")


# speedup vs baseline: 10.3098x; 10.3098x over previous
import jax
import jax.numpy as jnp
from jax.experimental import pallas as pl
from jax.experimental.pallas import tpu as pltpu

_B, _P, _NOTE, _D = 8, 2048, 512, 256
_VOCAB, _POS, _POSD = 100, 20, 10
_H = 256
_T = _NOTE
_ROWS = _T * _B


def _frontend_kernel(seq_col_ref, seq16_ref, ord_col_ref, snd_ref,
                     ew_ref, ep_ref, mw1_ref, mb1_ref, mw2_ref, mb2_ref,
                     enc_ref):
    P = _P
    seq = seq_col_ref[0]
    order = ord_col_ref[0]
    ew = ew_ref[...]
    ew = jnp.where(jax.lax.broadcasted_iota(jnp.int32, ew.shape, 0) == 0, 0.0, ew)
    oh_w = (seq == jax.lax.broadcasted_iota(jnp.int32, (1, _VOCAB), 1)).astype(jnp.float32)
    pe = jnp.dot(oh_w, ew, preferred_element_type=jnp.float32)
    ep = ep_ref[...]
    ep = jnp.where(jax.lax.broadcasted_iota(jnp.int32, ep.shape, 0) == 0, 0.0, ep)
    oh_p = (order == jax.lax.broadcasted_iota(jnp.int32, (1, _POS), 1)).astype(jnp.float32)
    ppe = jnp.dot(oh_p, ep, preferred_element_type=jnp.float32)
    x = jnp.concatenate([pe, ppe], axis=1)

    def _conv(xin, w_ref, b_ref, relu):
        z = jnp.zeros((1, xin.shape[1]), jnp.float32)
        xm = jnp.concatenate([z, xin[:-1]], axis=0)
        xp = jnp.concatenate([xin[1:], z], axis=0)
        y = (jnp.dot(xm, w_ref[0], preferred_element_type=jnp.float32)
             + jnp.dot(xin, w_ref[1], preferred_element_type=jnp.float32)
             + jnp.dot(xp, w_ref[2], preferred_element_type=jnp.float32))
        y = y + b_ref[...]
        return jnp.maximum(y, 0.0) if relu else y

    y1 = _conv(x, mw1_ref, mb1_ref, True)
    y2 = _conv(y1, mw2_ref, mb2_ref, False)

    seq16 = seq16_ref[0]
    r_i = jax.lax.broadcasted_iota(jnp.int32, (16, 128), 0)
    c_i = jax.lax.broadcasted_iota(jnp.int32, (16, 128), 1)
    pos = r_i * 128 + c_i
    m16 = ((seq16 > 1) & (pos < P - 1)).astype(jnp.float32)
    in_row_prev = jnp.concatenate([jnp.zeros((16, 1), jnp.float32), m16[:, :-1]], axis=1)
    carry = jnp.concatenate([jnp.zeros((1, 1), jnp.float32), m16[:-1, 127:128]], axis=0)
    prev = jnp.where(c_i == 0, carry, in_row_prev)
    starts = m16 * (1.0 - prev)
    ii = jax.lax.broadcasted_iota(jnp.int32, (128, 128), 0)
    jj = jax.lax.broadcasted_iota(jnp.int32, (128, 128), 1)
    tri = (ii <= jj).astype(jnp.float32)
    csum = jnp.dot(starts, tri, preferred_element_type=jnp.float32)
    rowtot = csum[:, 127:128]
    i2 = jax.lax.broadcasted_iota(jnp.int32, (16, 16), 0)
    j2 = jax.lax.broadcasted_iota(jnp.int32, (16, 16), 1)
    low = (j2 < i2).astype(jnp.float32)
    prefix = jnp.dot(low, rowtot, preferred_element_type=jnp.float32)
    runid = csum + prefix - 1.0
    runid = jnp.where(m16 > 0.5, runid, -1.0).astype(jnp.int32)

    n_iota = jax.lax.broadcasted_iota(jnp.int32, (_NOTE, 1), 0)
    y2aug = jnp.concatenate([y2, jnp.ones((P, 1), jnp.float32)], axis=1)
    acc = jnp.zeros((_NOTE, _D + 1), jnp.float32)
    for r in range(16):
        sel = (runid[r:r + 1, :] == n_iota).astype(jnp.float32)
        acc = acc + jnp.dot(sel, y2aug[r * 128:(r + 1) * 128],
                            preferred_element_type=jnp.float32)
    sums = acc[:, :_D]
    counts = acc[:, _D:_D + 1]
    agg = sums / jnp.maximum(counts, 1.0)
    snd = snd_ref[0]
    enc_ref[0] = jnp.concatenate([agg, snd, 1.0 / (snd + 1.0)], axis=1)


def _frontend(seq_col, seq16, ord_col, snd, ew, ep, mw1, mb1, mw2, mb2):
    def im_b(b):
        return (b, 0, 0)

    def im_w2(b):
        return (0, 0)

    def im_w3(b):
        return (0, 0, 0)

    return pl.pallas_call(
        _frontend_kernel,
        grid=(_B,),
        in_specs=[
            pl.BlockSpec((1, _P, 1), im_b),
            pl.BlockSpec((1, 16, 128), im_b),
            pl.BlockSpec((1, _P, 1), im_b),
            pl.BlockSpec((1, _NOTE, 1), im_b),
            pl.BlockSpec((_VOCAB, _D), im_w2),
            pl.BlockSpec((_POS, _POSD), im_w2),
            pl.BlockSpec((3, _D + _POSD, _D), im_w3),
            pl.BlockSpec((1, _D), im_w2),
            pl.BlockSpec((3, _D, _D), im_w3),
            pl.BlockSpec((1, _D), im_w2),
        ],
        out_specs=pl.BlockSpec((1, _NOTE, _D + 2), im_b),
        out_shape=jax.ShapeDtypeStruct((_B, _NOTE, _D + 2), jnp.float32),
    )(seq_col, seq16, ord_col, snd, ew, ep, mw1, mb1, mw2, mb2)


def _lstm_kernel(x_ref, wf_ref, bf_ref, wb_ref, bb_ref, whh_ref, out_ref,
                 xf_scr, xb_scr):
    xf_scr[...] = jnp.dot(x_ref[...], wf_ref[...],
                          preferred_element_type=jnp.float32) + bf_ref[...]
    xb_scr[...] = jnp.dot(x_ref[...], wb_ref[...],
                          preferred_element_type=jnp.float32) + bb_ref[...]
    whh = whh_ref[...]

    def step(t, carry):
        h, c = carry
        g = jnp.dot(h, whh, preferred_element_type=jnp.float32)
        tb = (_T - 1 - t) * _B
        gf = g[0:_B, 0:4 * _H] + xf_scr[pl.ds(t * _B, _B), :]
        gb = g[_B:2 * _B, 4 * _H:8 * _H] + xb_scr[pl.ds(tb, _B), :]
        gg = jnp.concatenate([gf, gb], axis=0)
        i = jax.nn.sigmoid(gg[:, 0:_H])
        f = jax.nn.sigmoid(gg[:, _H:2 * _H])
        gc = jnp.tanh(gg[:, 2 * _H:3 * _H])
        o = jax.nn.sigmoid(gg[:, 3 * _H:4 * _H])
        c = f * c + i * gc
        h = o * jnp.tanh(c)
        out_ref[pl.ds(t * _B, _B), 0:_H] = h[0:_B]
        out_ref[pl.ds(tb, _B), _H:2 * _H] = h[_B:2 * _B]
        return h, c

    jax.lax.fori_loop(0, _T, step,
                      (jnp.zeros((2 * _B, _H), jnp.float32),
                       jnp.zeros((2 * _B, _H), jnp.float32)))


def _lstm(x, wf, bf, wb, bb, whh):
    n_in = x.shape[1]
    return pl.pallas_call(
        _lstm_kernel,
        in_specs=[
            pl.BlockSpec((_ROWS, n_in), lambda: (0, 0)),
            pl.BlockSpec((n_in, 4 * _H), lambda: (0, 0)),
            pl.BlockSpec((1, 4 * _H), lambda: (0, 0)),
            pl.BlockSpec((n_in, 4 * _H), lambda: (0, 0)),
            pl.BlockSpec((1, 4 * _H), lambda: (0, 0)),
            pl.BlockSpec((_H, 8 * _H), lambda: (0, 0)),
        ],
        out_specs=pl.BlockSpec((_ROWS, 2 * _H), lambda: (0, 0)),
        out_shape=jax.ShapeDtypeStruct((_ROWS, 2 * _H), jnp.float32),
        scratch_shapes=[
            pltpu.VMEM((_ROWS, 4 * _H), jnp.float32),
            pltpu.VMEM((_ROWS, 4 * _H), jnp.float32),
        ],
        compiler_params=pltpu.CompilerParams(
            vmem_limit_bytes=100 * 1024 * 1024),
    )(x, wf, bf, wb, bb, whh)


def _outconv_kernel(x_ref, w1_ref, b1_ref, w2_ref, b2_ref, out_ref):
    x = x_ref[...]
    z = jnp.zeros((_B, x.shape[1]), jnp.float32)
    xm = jnp.concatenate([z, x[:-_B]], axis=0)
    xp = jnp.concatenate([x[_B:], z], axis=0)
    y = (jnp.dot(xm, w1_ref[0], preferred_element_type=jnp.float32)
         + jnp.dot(x, w1_ref[1], preferred_element_type=jnp.float32)
         + jnp.dot(xp, w1_ref[2], preferred_element_type=jnp.float32)
         + b1_ref[...])
    y = jnp.maximum(y, 0.0)
    z2 = jnp.zeros((_B, y.shape[1]), jnp.float32)
    ym = jnp.concatenate([z2, y[:-_B]], axis=0)
    yp = jnp.concatenate([y[_B:], z2], axis=0)
    out = (jnp.dot(ym, w2_ref[0], preferred_element_type=jnp.float32)
           + jnp.dot(y, w2_ref[1], preferred_element_type=jnp.float32)
           + jnp.dot(yp, w2_ref[2], preferred_element_type=jnp.float32)
           + b2_ref[...])
    out_ref[...] = out


def _outconv(x, w1, b1, w2, b2):
    return pl.pallas_call(
        _outconv_kernel,
        in_specs=[
            pl.BlockSpec((_ROWS, 2 * _H), lambda: (0, 0)),
            pl.BlockSpec((3, 2 * _H, _D), lambda: (0, 0, 0)),
            pl.BlockSpec((1, _D), lambda: (0, 0)),
            pl.BlockSpec((3, _D, 1), lambda: (0, 0, 0)),
            pl.BlockSpec((1, 1), lambda: (0, 0)),
        ],
        out_specs=pl.BlockSpec((_ROWS, 1), lambda: (0, 0)),
        out_shape=jax.ShapeDtypeStruct((_ROWS, 1), jnp.float32),
    )(x, w1, b1, w2, b2)


def kernel(score_note_dur, phoneme_seq, phoneme_order, emb_word, emb_pos,
           mix_w1, mix_b1, mix_w2, mix_b2,
           l0f_wih, l0f_whh, l0f_bih, l0f_bhh,
           l0b_wih, l0b_whh, l0b_bih, l0b_bhh,
           l1f_wih, l1f_whh, l1f_bih, l1f_bhh,
           l1b_wih, l1b_whh, l1b_bih, l1b_bhh,
           cnn_w1, cnn_b1, cnn_w2, cnn_b2):
    seq_col = phoneme_seq.astype(jnp.int32).reshape(_B, _P, 1)
    seq16 = phoneme_seq.astype(jnp.int32).reshape(_B, 16, 128)
    ord_col = phoneme_order.astype(jnp.int32).reshape(_B, _P, 1)
    snd = score_note_dur.reshape(_B, _NOTE, 1)
    mw1 = mix_w1.transpose(2, 1, 0)
    mw2 = mix_w2.transpose(2, 1, 0)
    enc = _frontend(seq_col, seq16, ord_col, snd, emb_word, emb_pos,
                    mw1, mix_b1.reshape(1, _D), mw2, mix_b2.reshape(1, _D))
    x0 = enc.transpose(1, 0, 2).reshape(_ROWS, _D + 2)
    out0 = _lstm(x0,
                 l0f_wih.T, (l0f_bih + l0f_bhh).reshape(1, 4 * _H),
                 l0b_wih.T, (l0b_bih + l0b_bhh).reshape(1, 4 * _H),
                 jnp.concatenate([l0f_whh.T, l0b_whh.T], axis=1))
    out1 = _lstm(out0,
                 l1f_wih.T, (l1f_bih + l1f_bhh).reshape(1, 4 * _H),
                 l1b_wih.T, (l1b_bih + l1b_bhh).reshape(1, 4 * _H),
                 jnp.concatenate([l1f_whh.T, l1b_whh.T], axis=1))
    y = _outconv(out1, cnn_w1.transpose(2, 1, 0), cnn_b1.reshape(1, _D),
                 cnn_w2.transpose(2, 1, 0), cnn_b2.reshape(1, 1))
    return y.reshape(_T, _B, 1).transpose(1, 0, 2)


# speedup vs baseline: 10.4279x; 1.0114x over previous
import jax
import jax.numpy as jnp
from jax.experimental import pallas as pl
from jax.experimental.pallas import tpu as pltpu

_B, _P, _NOTE, _D = 8, 2048, 512, 256
_VOCAB, _POS, _POSD = 100, 20, 10
_H = 256
_T = _NOTE
_ROWS = _T * _B


def _frontend_kernel(seq_col_ref, seq16_ref, ord_col_ref, snd_ref,
                     ew_ref, ep_ref, mw1_ref, mb1_ref, mw2_ref, mb2_ref,
                     enc_ref):
    P = _P
    seq = seq_col_ref[0]
    order = ord_col_ref[0]
    ew = ew_ref[...]
    ew = jnp.where(jax.lax.broadcasted_iota(jnp.int32, ew.shape, 0) == 0, 0.0, ew)
    oh_w = (seq == jax.lax.broadcasted_iota(jnp.int32, (1, _VOCAB), 1)).astype(jnp.float32)
    pe = jnp.dot(oh_w, ew, preferred_element_type=jnp.float32)
    ep = ep_ref[...]
    ep = jnp.where(jax.lax.broadcasted_iota(jnp.int32, ep.shape, 0) == 0, 0.0, ep)
    oh_p = (order == jax.lax.broadcasted_iota(jnp.int32, (1, _POS), 1)).astype(jnp.float32)
    ppe = jnp.dot(oh_p, ep, preferred_element_type=jnp.float32)
    x = jnp.concatenate([pe, ppe], axis=1)

    def _conv(xin, w_ref, b_ref, relu):
        z = jnp.zeros((1, xin.shape[1]), jnp.float32)
        xm = jnp.concatenate([z, xin[:-1]], axis=0)
        xp = jnp.concatenate([xin[1:], z], axis=0)
        y = (jnp.dot(xm, w_ref[0], preferred_element_type=jnp.float32)
             + jnp.dot(xin, w_ref[1], preferred_element_type=jnp.float32)
             + jnp.dot(xp, w_ref[2], preferred_element_type=jnp.float32))
        y = y + b_ref[...]
        return jnp.maximum(y, 0.0) if relu else y

    y1 = _conv(x, mw1_ref, mb1_ref, True)
    y2 = _conv(y1, mw2_ref, mb2_ref, False)

    seq16 = seq16_ref[0]
    r_i = jax.lax.broadcasted_iota(jnp.int32, (16, 128), 0)
    c_i = jax.lax.broadcasted_iota(jnp.int32, (16, 128), 1)
    pos = r_i * 128 + c_i
    m16 = ((seq16 > 1) & (pos < P - 1)).astype(jnp.float32)
    in_row_prev = jnp.concatenate([jnp.zeros((16, 1), jnp.float32), m16[:, :-1]], axis=1)
    carry = jnp.concatenate([jnp.zeros((1, 1), jnp.float32), m16[:-1, 127:128]], axis=0)
    prev = jnp.where(c_i == 0, carry, in_row_prev)
    starts = m16 * (1.0 - prev)
    ii = jax.lax.broadcasted_iota(jnp.int32, (128, 128), 0)
    jj = jax.lax.broadcasted_iota(jnp.int32, (128, 128), 1)
    tri = (ii <= jj).astype(jnp.float32)
    csum = jnp.dot(starts, tri, preferred_element_type=jnp.float32)
    rowtot = csum[:, 127:128]
    i2 = jax.lax.broadcasted_iota(jnp.int32, (16, 16), 0)
    j2 = jax.lax.broadcasted_iota(jnp.int32, (16, 16), 1)
    low = (j2 < i2).astype(jnp.float32)
    prefix = jnp.dot(low, rowtot, preferred_element_type=jnp.float32)
    runid = csum + prefix - 1.0
    runid = jnp.where(m16 > 0.5, runid, -1.0).astype(jnp.int32)

    n_iota = jax.lax.broadcasted_iota(jnp.int32, (_NOTE, 1), 0)
    y2aug = jnp.concatenate([y2, jnp.ones((P, 1), jnp.float32)], axis=1)
    acc = jnp.zeros((_NOTE, _D + 1), jnp.float32)
    for r in range(16):
        sel = (runid[r:r + 1, :] == n_iota).astype(jnp.float32)
        acc = acc + jnp.dot(sel, y2aug[r * 128:(r + 1) * 128],
                            preferred_element_type=jnp.float32)
    sums = acc[:, :_D]
    counts = acc[:, _D:_D + 1]
    agg = sums / jnp.maximum(counts, 1.0)
    snd = snd_ref[0]
    enc_ref[0] = jnp.concatenate([agg, snd, 1.0 / (snd + 1.0)], axis=1)


def _frontend(seq_col, seq16, ord_col, snd, ew, ep, mw1, mb1, mw2, mb2):
    def im_b(b):
        return (b, 0, 0)

    def im_w2(b):
        return (0, 0)

    def im_w3(b):
        return (0, 0, 0)

    return pl.pallas_call(
        _frontend_kernel,
        grid=(_B,),
        in_specs=[
            pl.BlockSpec((1, _P, 1), im_b),
            pl.BlockSpec((1, 16, 128), im_b),
            pl.BlockSpec((1, _P, 1), im_b),
            pl.BlockSpec((1, _NOTE, 1), im_b),
            pl.BlockSpec((_VOCAB, _D), im_w2),
            pl.BlockSpec((_POS, _POSD), im_w2),
            pl.BlockSpec((3, _D + _POSD, _D), im_w3),
            pl.BlockSpec((1, _D), im_w2),
            pl.BlockSpec((3, _D, _D), im_w3),
            pl.BlockSpec((1, _D), im_w2),
        ],
        out_specs=pl.BlockSpec((1, _NOTE, _D + 2), im_b),
        out_shape=jax.ShapeDtypeStruct((_B, _NOTE, _D + 2), jnp.float32),
    )(seq_col, seq16, ord_col, snd, ew, ep, mw1, mb1, mw2, mb2)


def _lstm_kernel(x_ref, wf_ref, bf_ref, wb_ref, bb_ref, whh_ref, out_ref,
                 xf_scr, xb_scr):
    xb16 = x_ref[...].astype(jnp.bfloat16)
    xf_scr[...] = jnp.dot(xb16, wf_ref[...],
                          preferred_element_type=jnp.float32) + bf_ref[...]
    xb_scr[...] = jnp.dot(xb16, wb_ref[...],
                          preferred_element_type=jnp.float32) + bb_ref[...]
    whh = whh_ref[...]

    def step(t, carry):
        h, c = carry
        g = jnp.dot(h.astype(jnp.bfloat16), whh,
                    preferred_element_type=jnp.float32)
        tb = (_T - 1 - t) * _B
        gf = g[0:_B, 0:4 * _H] + xf_scr[pl.ds(t * _B, _B), :]
        gb = g[_B:2 * _B, 4 * _H:8 * _H] + xb_scr[pl.ds(tb, _B), :]
        gg = jnp.concatenate([gf, gb], axis=0)
        i = jax.nn.sigmoid(gg[:, 0:_H])
        f = jax.nn.sigmoid(gg[:, _H:2 * _H])
        gc = jnp.tanh(gg[:, 2 * _H:3 * _H])
        o = jax.nn.sigmoid(gg[:, 3 * _H:4 * _H])
        c = f * c + i * gc
        h = o * jnp.tanh(c)
        out_ref[pl.ds(t * _B, _B), 0:_H] = h[0:_B]
        out_ref[pl.ds(tb, _B), _H:2 * _H] = h[_B:2 * _B]
        return h, c

    jax.lax.fori_loop(0, _T, step,
                      (jnp.zeros((2 * _B, _H), jnp.float32),
                       jnp.zeros((2 * _B, _H), jnp.float32)))


def _lstm(x, wf, bf, wb, bb, whh):
    wf = wf.astype(jnp.bfloat16)
    wb = wb.astype(jnp.bfloat16)
    whh = whh.astype(jnp.bfloat16)
    n_in = x.shape[1]
    return pl.pallas_call(
        _lstm_kernel,
        in_specs=[
            pl.BlockSpec((_ROWS, n_in), lambda: (0, 0)),
            pl.BlockSpec((n_in, 4 * _H), lambda: (0, 0)),
            pl.BlockSpec((1, 4 * _H), lambda: (0, 0)),
            pl.BlockSpec((n_in, 4 * _H), lambda: (0, 0)),
            pl.BlockSpec((1, 4 * _H), lambda: (0, 0)),
            pl.BlockSpec((_H, 8 * _H), lambda: (0, 0)),
        ],
        out_specs=pl.BlockSpec((_ROWS, 2 * _H), lambda: (0, 0)),
        out_shape=jax.ShapeDtypeStruct((_ROWS, 2 * _H), jnp.float32),
        scratch_shapes=[
            pltpu.VMEM((_ROWS, 4 * _H), jnp.float32),
            pltpu.VMEM((_ROWS, 4 * _H), jnp.float32),
        ],
        compiler_params=pltpu.CompilerParams(
            vmem_limit_bytes=100 * 1024 * 1024),
    )(x, wf, bf, wb, bb, whh)


def _outconv_kernel(x_ref, w1_ref, b1_ref, w2_ref, b2_ref, out_ref):
    x = x_ref[...]
    z = jnp.zeros((_B, x.shape[1]), jnp.float32)
    xm = jnp.concatenate([z, x[:-_B]], axis=0)
    xp = jnp.concatenate([x[_B:], z], axis=0)
    y = (jnp.dot(xm, w1_ref[0], preferred_element_type=jnp.float32)
         + jnp.dot(x, w1_ref[1], preferred_element_type=jnp.float32)
         + jnp.dot(xp, w1_ref[2], preferred_element_type=jnp.float32)
         + b1_ref[...])
    y = jnp.maximum(y, 0.0)
    z2 = jnp.zeros((_B, y.shape[1]), jnp.float32)
    ym = jnp.concatenate([z2, y[:-_B]], axis=0)
    yp = jnp.concatenate([y[_B:], z2], axis=0)
    out = (jnp.dot(ym, w2_ref[0], preferred_element_type=jnp.float32)
           + jnp.dot(y, w2_ref[1], preferred_element_type=jnp.float32)
           + jnp.dot(yp, w2_ref[2], preferred_element_type=jnp.float32)
           + b2_ref[...])
    out_ref[...] = out


def _outconv(x, w1, b1, w2, b2):
    return pl.pallas_call(
        _outconv_kernel,
        in_specs=[
            pl.BlockSpec((_ROWS, 2 * _H), lambda: (0, 0)),
            pl.BlockSpec((3, 2 * _H, _D), lambda: (0, 0, 0)),
            pl.BlockSpec((1, _D), lambda: (0, 0)),
            pl.BlockSpec((3, _D, 1), lambda: (0, 0, 0)),
            pl.BlockSpec((1, 1), lambda: (0, 0)),
        ],
        out_specs=pl.BlockSpec((_ROWS, 1), lambda: (0, 0)),
        out_shape=jax.ShapeDtypeStruct((_ROWS, 1), jnp.float32),
    )(x, w1, b1, w2, b2)


def kernel(score_note_dur, phoneme_seq, phoneme_order, emb_word, emb_pos,
           mix_w1, mix_b1, mix_w2, mix_b2,
           l0f_wih, l0f_whh, l0f_bih, l0f_bhh,
           l0b_wih, l0b_whh, l0b_bih, l0b_bhh,
           l1f_wih, l1f_whh, l1f_bih, l1f_bhh,
           l1b_wih, l1b_whh, l1b_bih, l1b_bhh,
           cnn_w1, cnn_b1, cnn_w2, cnn_b2):
    seq_col = phoneme_seq.astype(jnp.int32).reshape(_B, _P, 1)
    seq16 = phoneme_seq.astype(jnp.int32).reshape(_B, 16, 128)
    ord_col = phoneme_order.astype(jnp.int32).reshape(_B, _P, 1)
    snd = score_note_dur.reshape(_B, _NOTE, 1)
    mw1 = mix_w1.transpose(2, 1, 0)
    mw2 = mix_w2.transpose(2, 1, 0)
    enc = _frontend(seq_col, seq16, ord_col, snd, emb_word, emb_pos,
                    mw1, mix_b1.reshape(1, _D), mw2, mix_b2.reshape(1, _D))
    x0 = enc.transpose(1, 0, 2).reshape(_ROWS, _D + 2)
    out0 = _lstm(x0,
                 l0f_wih.T, (l0f_bih + l0f_bhh).reshape(1, 4 * _H),
                 l0b_wih.T, (l0b_bih + l0b_bhh).reshape(1, 4 * _H),
                 jnp.concatenate([l0f_whh.T, l0b_whh.T], axis=1))
    out1 = _lstm(out0,
                 l1f_wih.T, (l1f_bih + l1f_bhh).reshape(1, 4 * _H),
                 l1b_wih.T, (l1b_bih + l1b_bhh).reshape(1, 4 * _H),
                 jnp.concatenate([l1f_whh.T, l1b_whh.T], axis=1))
    y = _outconv(out1, cnn_w1.transpose(2, 1, 0), cnn_b1.reshape(1, _D),
                 cnn_w2.transpose(2, 1, 0), cnn_b2.reshape(1, 1))
    return y.reshape(_T, _B, 1).transpose(1, 0, 2)
